# baseline (speedup 1.0000x reference)
import jax
import jax.numpy as jnp
from jax.experimental import pallas as pl
from jax.experimental.pallas import tpu as pltpu


def _se_body(x_ref, w1t_ref, w2t_ref, o_ref):
    y = jnp.sum(x_ref[...], axis=(2, 3))
    h = jnp.dot(y, w1t_ref[...], preferred_element_type=jnp.float32)
    h = jnp.maximum(h, 0.0)
    logits = jnp.dot(h, w2t_ref[...], preferred_element_type=jnp.float32)

    m = jnp.max(logits, axis=-1, keepdims=True)
    e = jnp.exp(logits - m)
    probs = e * pl.reciprocal(jnp.sum(e, axis=-1, keepdims=True),
                              approx=False)
    o_ref[...] = probs[None]


def _se_layer(x, w1, w2):
    b, c, h, w = x.shape
    hid, c_in = w1.shape
    out_ch, hid2 = w2.shape
    assert c_in == c and hid2 == hid

    w_pad = max(128, -(-w // 128) * 128)
    h_pad = max(8, -(-h // 8) * 8)
    slab_bytes = c * h_pad * w_pad * 4
    budget = 15 << 20
    tb = b
    for d in range(b, 0, -1):
        if b % d == 0 and d * slab_bytes <= budget:
            tb = d
            break
    nb = b // tb

    w1t = jnp.asarray(w1).T * (1.0 / (h * w))
    w2t = jnp.asarray(w2).T

    vmem_limit = min(2 * tb * slab_bytes + (4 << 20), 56 << 20)

    out = pl.pallas_call(
        _se_body,
        out_shape=jax.ShapeDtypeStruct((nb, tb, out_ch), jnp.float32),
        grid=(nb,),
        in_specs=[
            pl.BlockSpec((tb, c, h, w), lambda i: (i, 0, 0, 0)),
            pl.BlockSpec((c, hid), lambda i: (0, 0)),
            pl.BlockSpec((hid, out_ch), lambda i: (0, 0)),
        ],
        out_specs=pl.BlockSpec((1, tb, out_ch), lambda i: (i, 0, 0)),
        compiler_params=pltpu.CompilerParams(
            dimension_semantics=("parallel",),
            vmem_limit_bytes=vmem_limit,
        ),
    )(x, w1t, w2t)

    return out.reshape(b, out_ch, 1, 1)


def kernel(x, w1, w2):
    return _se_layer(x, w1, w2)


# speedup vs baseline: 1.7653x; 1.7653x over previous
import jax
import jax.numpy as jnp
from jax.experimental import pallas as pl
from jax.experimental.pallas import tpu as pltpu


def _se_body(x_ref, w1t_ref, w2t_ref, o_ref):
    y = jnp.sum(x_ref[...], axis=-1)
    h = jnp.dot(y, w1t_ref[...], preferred_element_type=jnp.float32)
    h = jnp.maximum(h, 0.0)
    logits = jnp.dot(h, w2t_ref[...], preferred_element_type=jnp.float32)

    m = jnp.max(logits, axis=-1, keepdims=True)
    e = jnp.exp(logits - m)
    probs = e * pl.reciprocal(jnp.sum(e, axis=-1, keepdims=True),
                              approx=False)
    o_ref[...] = probs[None]


def _se_layer(x, w1, w2):
    b, c, h, w = x.shape
    hid, c_in = w1.shape
    out_ch, hid2 = w2.shape
    assert c_in == c and hid2 == hid

    hw = h * w
    x_flat = x.reshape(b, c, hw)

    tb = 8 if (b % 8 == 0 and b > 8) else b
    nb = b // tb
    slab_bytes = c * hw * 4

    w1t = jnp.asarray(w1).T * (1.0 / (h * w))
    w2t = jnp.asarray(w2).T

    vmem_limit = min(2 * tb * slab_bytes + (4 << 20), 56 << 20)

    out = pl.pallas_call(
        _se_body,
        out_shape=jax.ShapeDtypeStruct((nb, tb, out_ch), jnp.float32),
        grid=(nb,),
        in_specs=[
            pl.BlockSpec((tb, c, hw), lambda i: (i, 0, 0)),
            pl.BlockSpec((c, hid), lambda i: (0, 0)),
            pl.BlockSpec((hid, out_ch), lambda i: (0, 0)),
        ],
        out_specs=pl.BlockSpec((1, tb, out_ch), lambda i: (i, 0, 0)),
        compiler_params=pltpu.CompilerParams(
            dimension_semantics=("parallel",),
            vmem_limit_bytes=vmem_limit,
        ),
    )(x_flat, w1t, w2t)

    return out.reshape(b, out_ch, 1, 1)


def kernel(x, w1, w2):
    return _se_layer(x, w1, w2)
